# baseline (speedup 1.0000x reference)
import jax
import jax.numpy as jnp
from jax.experimental import pallas as pl
from jax.experimental.pallas import tpu as pltpu

B, L, D = 4096, 200, 128
BB = 64


def _body(ids_ref, ids3_ref, pos_ref, memb_ref, out_ref, lab_ref, mask_ref):
    ids = ids_ref[...]
    lane = jax.lax.broadcasted_iota(jnp.int32, (BB, L), 1)
    labels = jnp.where(lane == (L - 1), 0, jnp.roll(ids, -1, axis=1))
    lab_ref[...] = labels
    mask_ref[...] = labels != 0

    ids3 = ids3_ref[...]
    sub = jax.lax.broadcasted_iota(jnp.int32, (BB, L, 1), 1)
    labels3 = jnp.where(sub == (L - 1), 0, jnp.roll(ids3, -1, axis=1))
    memb = memb_ref[...]
    out_ref[...] = jnp.where(labels3 != 0, pos_ref[...], memb)


def kernel(pos_emb, itemid_seq, training, masked_item_embedding):
    del training
    ids3 = itemid_seq.reshape(B, L, 1)
    memb3 = masked_item_embedding.reshape(1, 1, D)
    grid = (B // BB,)
    out, labels, mask = pl.pallas_call(
        _body,
        grid=grid,
        in_specs=[
            pl.BlockSpec((BB, L), lambda i: (i, 0)),
            pl.BlockSpec((BB, L, 1), lambda i: (i, 0, 0)),
            pl.BlockSpec((BB, L, D), lambda i: (i, 0, 0)),
            pl.BlockSpec((1, 1, D), lambda i: (0, 0, 0)),
        ],
        out_specs=[
            pl.BlockSpec((BB, L, D), lambda i: (i, 0, 0)),
            pl.BlockSpec((BB, L), lambda i: (i, 0)),
            pl.BlockSpec((BB, L), lambda i: (i, 0)),
        ],
        out_shape=[
            jax.ShapeDtypeStruct((B, L, D), jnp.float32),
            jax.ShapeDtypeStruct((B, L), jnp.int32),
            jax.ShapeDtypeStruct((B, L), jnp.bool_),
        ],
    )(itemid_seq, ids3, pos_emb, memb3)
    return (out, labels, mask)


# speedup vs baseline: 1.8953x; 1.8953x over previous
import jax
import jax.numpy as jnp
from jax.experimental import pallas as pl
from jax.experimental.pallas import tpu as pltpu

B, L, D = 4096, 200, 128
BB = 64


def _body(ids_ref, pos_ref, memb_ref, out_ref, lab_ref, mask_ref):
    ids = ids_ref[...]
    lane = jax.lax.broadcasted_iota(jnp.int32, (BB, L), 1)
    labels = jnp.where(lane == (L - 1), 0, jnp.roll(ids, -1, axis=1))
    lab_ref[...] = labels
    mask_ref[...] = labels != 0

    labels3 = jnp.transpose(labels.reshape(BB, 1, L), (0, 2, 1))
    memb = memb_ref[...]
    out_ref[...] = jnp.where(labels3 != 0, pos_ref[...], memb)


def kernel(pos_emb, itemid_seq, training, masked_item_embedding):
    del training
    memb3 = masked_item_embedding.reshape(1, 1, D)
    grid = (B // BB,)
    out, labels, mask = pl.pallas_call(
        _body,
        grid=grid,
        in_specs=[
            pl.BlockSpec((BB, L), lambda i: (i, 0)),
            pl.BlockSpec((BB, L, D), lambda i: (i, 0, 0)),
            pl.BlockSpec((1, 1, D), lambda i: (0, 0, 0)),
        ],
        out_specs=[
            pl.BlockSpec((BB, L, D), lambda i: (i, 0, 0)),
            pl.BlockSpec((BB, L), lambda i: (i, 0)),
            pl.BlockSpec((BB, L), lambda i: (i, 0)),
        ],
        out_shape=[
            jax.ShapeDtypeStruct((B, L, D), jnp.float32),
            jax.ShapeDtypeStruct((B, L), jnp.int32),
            jax.ShapeDtypeStruct((B, L), jnp.bool_),
        ],
    )(itemid_seq, pos_emb, memb3)
    return (out, labels, mask)
